# baseline (speedup 1.0000x reference)
import functools

import jax
import jax.numpy as jnp
from jax import lax
from jax.experimental import pallas as pl
from jax.experimental.pallas import tpu as pltpu
from jax.experimental.pallas import tpu_sc as plsc

B, L = 16384, 50
BL = B * L
D_DAY, D_TIME, D_LOC = 32, 32, 128
D_OUT = D_DAY + D_TIME + 2 * D_LOC
N_DAY, N_TIME, N_LOC = 7, 48, 201

NUM_CORES = 2
NUM_SUBCORES = 16
NW = NUM_CORES * NUM_SUBCORES
ELEMS_PER_W = B // NW
ROWS_PER_W = ELEMS_PER_W * L

IDX_BLK_ELEMS = 20
IDXBLK = IDX_BLK_ELEMS * L


def _sc_embed(day_i, time_i, locx_i, locy_i, day_table, time_table,
              locx_table, locy_table):
  mesh = plsc.VectorSubcoreMesh(core_axis_name="c", subcore_axis_name="s")

  @functools.partial(
      pl.kernel,
      mesh=mesh,
      out_type=jax.ShapeDtypeStruct((B, L, D_OUT), jnp.float32),
      scratch_types=[
          pltpu.VMEM((IDXBLK,), jnp.int32),
          pltpu.VMEM((IDXBLK,), jnp.int32),
          pltpu.VMEM((IDXBLK,), jnp.int32),
          pltpu.VMEM((IDXBLK,), jnp.int32),
          pltpu.VMEM((N_DAY * D_DAY,), jnp.float32),
          pltpu.VMEM((N_TIME * D_TIME,), jnp.float32),
          pltpu.VMEM((N_LOC * D_LOC,), jnp.float32),
          pltpu.VMEM((N_LOC * D_LOC,), jnp.float32),
          pltpu.VMEM((2, L, D_OUT), jnp.float32),
          pltpu.SemaphoreType.DMA,
          pltpu.SemaphoreType.DMA,
      ],
  )
  def k(day_h, time_h, locx_h, locy_h, dt_h, tt_h, xt_h, yt_h, out_h,
        di_v, ti_v, xi_v, yi_v, dt_v, tt_v, xt_v, yt_v, asm_v, w0, w1):
    wid = lax.axis_index("s") * NUM_CORES + lax.axis_index("c")
    e_base = wid * ELEMS_PER_W
    wsems = (w0, w1)

    pltpu.sync_copy(dt_h, dt_v)
    pltpu.sync_copy(tt_h, tt_v)
    pltpu.sync_copy(xt_h, xt_v)
    pltpu.sync_copy(yt_h, yt_v)

    def load_idx_block(g):
      base = e_base * L + g * IDXBLK
      pltpu.sync_copy(day_h.at[pl.ds(base, IDXBLK)], di_v)
      pltpu.sync_copy(time_h.at[pl.ds(base, IDXBLK)], ti_v)
      pltpu.sync_copy(locx_h.at[pl.ds(base, IDXBLK)], xi_v)
      pltpu.sync_copy(locy_h.at[pl.ds(base, IDXBLK)], yi_v)

    def write_copy(e, a):
      return pltpu.make_async_copy(asm_v.at[a], out_h.at[e_base + e],
                                   wsems[a])

    def assemble(e, a):
      off = lax.rem(e, IDX_BLK_ELEMS) * L

      def rows(r0, n, idx_off):
        dv = di_v[pl.ds(off + idx_off, 16)]
        tv = ti_v[pl.ds(off + idx_off, 16)]
        xv = xi_v[pl.ds(off + idx_off, 16)]
        yv = yi_v[pl.ds(off + idx_off, 16)]
        pv = (dv << 22) | (tv << 16) | (xv << 8) | yv
        for rr in range(n):
          r = r0 + rr
          lane = r - idx_off
          p = pv[lane]
          s_d = (p >> 22) << 5
          s_t = ((p >> 16) & 63) << 5
          s_x = ((p >> 8) & 255) << 7
          s_y = (p & 255) << 7
          for j in range(D_DAY // 16):
            asm_v[a, r, pl.ds(16 * j, 16)] = dt_v[pl.ds(s_d + 16 * j, 16)]
          for j in range(D_TIME // 16):
            asm_v[a, r, pl.ds(D_DAY + 16 * j, 16)] = (
                tt_v[pl.ds(s_t + 16 * j, 16)])
          for j in range(D_LOC // 16):
            asm_v[a, r, pl.ds(D_DAY + D_TIME + 16 * j, 16)] = (
                xt_v[pl.ds(s_x + 16 * j, 16)])
          for j in range(D_LOC // 16):
            asm_v[a, r, pl.ds(D_DAY + D_TIME + D_LOC + 16 * j, 16)] = (
                yt_v[pl.ds(s_y + 16 * j, 16)])

      rows(0, 16, 0)
      rows(16, 16, 16)
      rows(32, 16, 32)
      rows(48, 2, 34)

    load_idx_block(0)

    def step(e, a):
      @pl.when(lax.rem(e, IDX_BLK_ELEMS) == 0)
      def _():
        @pl.when(e > 0)
        def _():
          load_idx_block(e // IDX_BLK_ELEMS)

      @pl.when(e >= 2)
      def _():
        write_copy(e - 2, a).wait()

      assemble(e, a)
      write_copy(e, a).start()

    def body(i, carry):
      e0 = 2 * i
      step(e0, 0)
      step(e0 + 1, 1)
      return carry

    lax.fori_loop(0, ELEMS_PER_W // 2, body, 0)

    write_copy(ELEMS_PER_W - 2, 0).wait()
    write_copy(ELEMS_PER_W - 1, 1).wait()

  return k(day_i, time_i, locx_i, locy_i,
           day_table.reshape(N_DAY * D_DAY),
           time_table.reshape(N_TIME * D_TIME),
           locx_table.reshape(N_LOC * D_LOC),
           locy_table.reshape(N_LOC * D_LOC))


def kernel(day, time, location_x, location_y, day_table, time_table,
           locx_table, locy_table):
  day_i = day.reshape(BL).astype(jnp.int32)
  time_i = time.reshape(BL).astype(jnp.int32)
  locx_i = location_x.reshape(BL).astype(jnp.int32)
  locy_i = location_y.reshape(BL).astype(jnp.int32)
  return _sc_embed(day_i, time_i, locx_i, locy_i, day_table, time_table,
                   locx_table, locy_table)


# speedup vs baseline: 1.8034x; 1.8034x over previous
import functools

import jax
import jax.numpy as jnp
from jax import lax
from jax.experimental import pallas as pl
from jax.experimental.pallas import tpu as pltpu
from jax.experimental.pallas import tpu_sc as plsc

B, L = 16384, 50
BL = B * L
D_DAY, D_TIME, D_LOC = 32, 32, 128
D_OUT = D_DAY + D_TIME + 2 * D_LOC
N_DAY, N_TIME, N_LOC = 7, 48, 201

NUM_CORES = 2
NUM_SUBCORES = 16
NW = NUM_CORES * NUM_SUBCORES
ELEMS_PER_W = B // NW
ROWS_PER_W = ELEMS_PER_W * L

IDX_BLK_ELEMS = 20
IDXBLK = IDX_BLK_ELEMS * L


def _sc_embed(day_i, time_i, locx_i, locy_i, day_table, time_table,
              locx_table, locy_table):
  mesh = plsc.VectorSubcoreMesh(core_axis_name="c", subcore_axis_name="s")

  @functools.partial(
      pl.kernel,
      mesh=mesh,
      out_type=jax.ShapeDtypeStruct((B, L, D_OUT), jnp.float32),
      scratch_types=[
          pltpu.VMEM((IDXBLK,), jnp.int32),
          pltpu.VMEM((IDXBLK,), jnp.int32),
          pltpu.VMEM((IDXBLK,), jnp.int32),
          pltpu.VMEM((IDXBLK,), jnp.int32),
          pltpu.VMEM((N_DAY * D_DAY,), jnp.float32),
          pltpu.VMEM((N_TIME * D_TIME,), jnp.float32),
          pltpu.VMEM((N_LOC * D_LOC,), jnp.float32),
          pltpu.VMEM((N_LOC * D_LOC,), jnp.float32),
          pltpu.VMEM((2, L, D_OUT), jnp.float32),
          pltpu.SemaphoreType.DMA,
          pltpu.SemaphoreType.DMA,
      ],
  )
  def k(day_h, time_h, locx_h, locy_h, dt_h, tt_h, xt_h, yt_h, out_h,
        di_v, ti_v, xi_v, yi_v, dt_v, tt_v, xt_v, yt_v, asm_v, w0, w1):
    wid = lax.axis_index("s") * NUM_CORES + lax.axis_index("c")
    e_base = wid * ELEMS_PER_W
    wsems = (w0, w1)

    pltpu.sync_copy(dt_h, dt_v)
    pltpu.sync_copy(tt_h, tt_v)
    pltpu.sync_copy(xt_h, xt_v)
    pltpu.sync_copy(yt_h, yt_v)

    def load_idx_block(g):
      base = e_base * L + g * IDXBLK
      pltpu.sync_copy(day_h.at[pl.ds(base, IDXBLK)], di_v)
      pltpu.sync_copy(time_h.at[pl.ds(base, IDXBLK)], ti_v)
      pltpu.sync_copy(locx_h.at[pl.ds(base, IDXBLK)], xi_v)
      pltpu.sync_copy(locy_h.at[pl.ds(base, IDXBLK)], yi_v)

    def write_copy(e, a):
      return pltpu.make_async_copy(asm_v.at[a], out_h.at[e_base + e],
                                   wsems[a])

    SEGS = ([(16 * j, 0) for j in range(D_DAY // 16)] +
            [(D_DAY + 16 * j, 1) for j in range(D_TIME // 16)] +
            [(D_DAY + D_TIME + 16 * j, 2) for j in range(D_LOC // 16)] +
            [(D_DAY + D_TIME + D_LOC + 16 * j, 3)
             for j in range(D_LOC // 16)])

    def assemble(e, a):
      off = lax.rem(e, IDX_BLK_ELEMS) * L
      tabs = (dt_v, tt_v, xt_v, yt_v)

      def packed_vec(idx_off):
        dv = di_v[pl.ds(off + idx_off, 16)]
        tv = ti_v[pl.ds(off + idx_off, 16)]
        xv = xi_v[pl.ds(off + idx_off, 16)]
        yv = yi_v[pl.ds(off + idx_off, 16)]
        return (dv << 22) | (tv << 16) | (xv << 8) | yv

      lane_of = [(r // 16 * 16, r % 16) for r in range(48)] + [(34, 14),
                                                              (34, 15)]
      pvs = {}

      def extract(r):
        g, lane = lane_of[r]
        if g not in pvs:
          pvs[g] = packed_vec(g)
        return pvs[g][lane]

      def unpack(p):
        return ((p >> 22) << 5, ((p >> 16) & 63) << 5,
                ((p >> 8) & 255) << 7, (p & 255) << 7)

      COL0 = (0, D_DAY, D_DAY + D_TIME, D_DAY + D_TIME + D_LOC)

      p_cur = extract(0)
      prev_vals = None
      for r in range(L):
        offs = unpack(p_cur)
        if r + 1 < L:
          p_cur = extract(r + 1)
        new_vals = []
        for i, (col, t) in enumerate(SEGS):
          new_vals.append(tabs[t][pl.ds(offs[t] + col - COL0[t], 16)])
          if prev_vals is not None:
            asm_v[a, r - 1, pl.ds(col, 16)] = prev_vals[i]
        prev_vals = new_vals
      for i, (col, t) in enumerate(SEGS):
        asm_v[a, L - 1, pl.ds(col, 16)] = prev_vals[i]

    load_idx_block(0)

    def step(e, a):
      @pl.when(lax.rem(e, IDX_BLK_ELEMS) == 0)
      def _():
        @pl.when(e > 0)
        def _():
          load_idx_block(e // IDX_BLK_ELEMS)

      @pl.when(e >= 2)
      def _():
        write_copy(e - 2, a).wait()

      assemble(e, a)
      write_copy(e, a).start()

    def body(i, carry):
      e0 = 2 * i
      step(e0, 0)
      step(e0 + 1, 1)
      return carry

    lax.fori_loop(0, ELEMS_PER_W // 2, body, 0)

    write_copy(ELEMS_PER_W - 2, 0).wait()
    write_copy(ELEMS_PER_W - 1, 1).wait()

  return k(day_i, time_i, locx_i, locy_i,
           day_table.reshape(N_DAY * D_DAY),
           time_table.reshape(N_TIME * D_TIME),
           locx_table.reshape(N_LOC * D_LOC),
           locy_table.reshape(N_LOC * D_LOC))


def kernel(day, time, location_x, location_y, day_table, time_table,
           locx_table, locy_table):
  day_i = day.reshape(BL).astype(jnp.int32)
  time_i = time.reshape(BL).astype(jnp.int32)
  locx_i = location_x.reshape(BL).astype(jnp.int32)
  locy_i = location_y.reshape(BL).astype(jnp.int32)
  return _sc_embed(day_i, time_i, locx_i, locy_i, day_table, time_table,
                   locx_table, locy_table)
